# baseline (speedup 1.0000x reference)
import jax
import jax.numpy as jnp
from jax import lax
from jax.experimental import pallas as pl
from jax.experimental.pallas import tpu as pltpu
from jax.experimental.pallas import tpu_sc as plsc

_PAD = 0
_MASK = 1
_B, _S, _D = 4096, 200, 64
_NC, _NS, _L = 2, 16, 16
_NW = _NC * _NS
_BPW = _B // _NW
_CB = 2
_CHUNK = _CB * _S
_SUB = 100
_NSUB = _CHUNK // _SUB
_STEPS = _BPW // _CB
_GROUPS = _CHUNK // _L


def _body(ids_hbm, table_hbm, pm_hbm, out_hbm,
          ids_v0, ids_v1, idx_v0, idx_v1, rows_v0, rows_v1, pm_v,
          ids_sem, gat_sem, out_sem0, out_sem1):
    ids_bufs = (ids_v0, ids_v1)
    idx_bufs = (idx_v0, idx_v1)
    rows_bufs = (rows_v0, rows_v1)
    out_sems = (out_sem0, out_sem1)

    wid = lax.axis_index("s") * _NC + lax.axis_index("c")
    b_base = wid * _BPW

    def ids_src(i):
        return ids_hbm.at[pl.ds(b_base + i * _CB, _CB), :]

    def out_dst(i):
        return out_hbm.at[pl.ds(b_base + i * _CB, _CB), :, :]

    iota16 = lax.iota(jnp.int32, _L)

    pltpu.sync_copy(pm_hbm, pm_v)
    pltpu.async_copy(ids_src(0), ids_bufs[0], ids_sem)

    def pair(i2, carry):
        for b in range(2):
            i = 2 * i2 + b
            pltpu.make_async_copy(ids_src(i), ids_bufs[b], ids_sem).wait()

            @pl.when(i < _STEPS - 1)
            def _prefetch():
                pltpu.async_copy(ids_src(i + 1), ids_bufs[1 - b], ids_sem)

            def repack(g, acc):
                n = g * _L + iota16
                v_ids = plsc.load_gather(ids_bufs[b], [n // _S, n % _S])
                plsc.store_scatter(idx_bufs[b], [n // _SUB, n % _SUB], v_ids)
                return acc | jnp.where(v_ids < 2, 1, 0).astype(jnp.int32)

            spec = lax.fori_loop(0, _GROUPS, repack,
                                 jnp.zeros((_L,), jnp.int32))
            cnt = plsc.all_reduce_population_count(spec > 0)

            @pl.when(i2 > 0)
            def _drain():
                pltpu.make_async_copy(rows_bufs[b], out_dst(i),
                                      out_sems[b]).wait()

            copies = []
            for j in range(_NSUB):
                copies.append(pltpu.async_copy(
                    table_hbm.at[idx_bufs[b].at[j]],
                    rows_bufs[b].at[(j * _SUB) // _S,
                                    pl.ds((j * _SUB) % _S, _SUB), :],
                    gat_sem))
            for c in copies:
                c.wait()

            @pl.when(cnt[0] > 0)
            def _fixup():
                def fix_group(g, carry2):
                    n = g * _L + iota16
                    v_ids = plsc.load_gather(idx_bufs[b],
                                             [n // _SUB, n % _SUB])
                    special = v_ids < 2
                    sel = jnp.where(v_ids == _MASK, 1, 0).astype(jnp.int32)
                    rb = n // _S
                    rs = n % _S
                    for d in range(_D):
                        dcol = jnp.full((_L,), d, jnp.int32)
                        val = plsc.load_gather(pm_v, [sel, dcol])
                        plsc.store_scatter(rows_bufs[b], [rb, rs, dcol], val,
                                           mask=special)
                    return carry2

                lax.fori_loop(0, _GROUPS, fix_group, 0)

            pltpu.async_copy(rows_bufs[b], out_dst(i), out_sems[b])
        return carry

    lax.fori_loop(0, _STEPS // 2, pair, 0)

    for b in range(2):
        pltpu.make_async_copy(rows_bufs[b], out_dst(_STEPS - 2 + b),
                              out_sems[b]).wait()


def kernel(road_ids, road_table, pad_emb, mask_emb):
    pm = jnp.stack([pad_emb, mask_emb])

    mesh = plsc.VectorSubcoreMesh(core_axis_name="c", subcore_axis_name="s")
    run = pl.kernel(
        _body,
        mesh=mesh,
        compiler_params=pltpu.CompilerParams(needs_layout_passes=False,
                                             use_tc_tiling_on_sc=False),
        out_type=jax.ShapeDtypeStruct((_B, _S, _D), jnp.float32),
        scratch_types=[
            pltpu.VMEM((_CB, _S), jnp.int32),
            pltpu.VMEM((_CB, _S), jnp.int32),
            pltpu.VMEM((_NSUB, _SUB), jnp.int32),
            pltpu.VMEM((_NSUB, _SUB), jnp.int32),
            pltpu.VMEM((_CB, _S, _D), jnp.float32),
            pltpu.VMEM((_CB, _S, _D), jnp.float32),
            pltpu.VMEM((2, _D), jnp.float32),
            pltpu.SemaphoreType.DMA,
            pltpu.SemaphoreType.DMA,
            pltpu.SemaphoreType.DMA,
            pltpu.SemaphoreType.DMA,
        ],
    )
    return run(road_ids, road_table, pm)


# speedup vs baseline: 1.2317x; 1.2317x over previous
import jax
import jax.numpy as jnp
from jax import lax
from jax.experimental import pallas as pl
from jax.experimental.pallas import tpu as pltpu
from jax.experimental.pallas import tpu_sc as plsc

_PAD = 0
_MASK = 1
_B, _S, _D = 4096, 200, 64
_DP = 128
_NC, _NS, _L = 2, 16, 16
_NW = _NC * _NS
_BPW = _B // _NW
_CB = 2
_CHUNK = _CB * _S
_SUB = 100
_NSUB = _CHUNK // _SUB
_STEPS = _BPW // _CB
_GROUPS = _CHUNK // _L


def _body(ids_hbm, table_hbm, pm_hbm, out_hbm,
          ids_v0, ids_v1, idx_v0, idx_v1, rows_v0, rows_v1, pm_v,
          ids_sem, gat_sem0, gat_sem1, out_sem0, out_sem1):
    ids_bufs = (ids_v0, ids_v1)
    idx_bufs = (idx_v0, idx_v1)
    rows_bufs = (rows_v0, rows_v1)
    gat_sems = (gat_sem0, gat_sem1)
    out_sems = (out_sem0, out_sem1)

    wid = lax.axis_index("s") * _NC + lax.axis_index("c")
    b_base = wid * _BPW

    def ids_src(i):
        return ids_hbm.at[pl.ds(b_base + jnp.minimum(i, _STEPS - 1) * _CB,
                                _CB), :]

    def out_dst(i):
        return out_hbm.at[pl.ds(b_base + i * _CB, _CB), :, :]

    iota16 = lax.iota(jnp.int32, _L)

    def repack(buf):
        def step_g(g, acc):
            n = g * _L + iota16
            v_ids = plsc.load_gather(ids_bufs[buf], [n // _S, n % _S])
            plsc.store_scatter(idx_bufs[buf], [n // _SUB, n % _SUB], v_ids)
            return acc | jnp.where(v_ids < 2, 1, 0).astype(jnp.int32)

        spec = lax.fori_loop(0, _GROUPS, step_g, jnp.zeros((_L,), jnp.int32))
        return plsc.all_reduce_population_count(spec > 0)[0]

    def gather_copies(buf, make_only):
        mk = pltpu.make_async_copy if make_only else pltpu.async_copy
        return [
            mk(table_hbm.at[idx_bufs[buf].at[j]],
               rows_bufs[buf].at[(j * _SUB) // _S,
                                 pl.ds((j * _SUB) % _S, _SUB), :],
               gat_sems[buf])
            for j in range(_NSUB)
        ]

    def fixup(buf):
        def fix_group(g, carry2):
            n = g * _L + iota16
            v_ids = plsc.load_gather(idx_bufs[buf], [n // _SUB, n % _SUB])
            special = v_ids < 2
            sel = jnp.where(v_ids == _MASK, 1, 0).astype(jnp.int32)
            rb = n // _S
            rs = n % _S
            for d in range(_D):
                dcol = jnp.full((_L,), d, jnp.int32)
                val = plsc.load_gather(pm_v, [sel, dcol])
                plsc.store_scatter(rows_bufs[buf], [rb, rs, dcol], val,
                                   mask=special)
            return carry2

        lax.fori_loop(0, _GROUPS, fix_group, 0)

    pltpu.sync_copy(pm_hbm, pm_v)
    pltpu.sync_copy(ids_src(0), ids_bufs[0])
    pltpu.async_copy(ids_src(1), ids_bufs[1], ids_sem)
    cnt0 = repack(0)
    gather_copies(0, make_only=False)

    def pair(i2, cnt_cur):
        for b in range(2):
            i = 2 * i2 + b
            nxt = 1 - b
            pltpu.make_async_copy(ids_src(i + 1), ids_bufs[nxt],
                                  ids_sem).wait()
            pltpu.async_copy(ids_src(i + 2), ids_bufs[b], ids_sem)
            cnt_nxt = repack(nxt)

            @pl.when(i >= 1)
            def _drain():
                pltpu.make_async_copy(rows_bufs[nxt], out_dst(i - 1),
                                      out_sems[nxt]).wait()

            gather_copies(nxt, make_only=False)

            for c in gather_copies(b, make_only=True):
                c.wait()

            @pl.when(cnt_cur > 0)
            def _fix():
                fixup(b)

            pltpu.async_copy(rows_bufs[b], out_dst(i), out_sems[b])
            cnt_cur = cnt_nxt
        return cnt_cur

    lax.fori_loop(0, _STEPS // 2, pair, cnt0)

    pltpu.make_async_copy(ids_src(_STEPS - 1), ids_bufs[0], ids_sem).wait()
    for c in gather_copies(0, make_only=True):
        c.wait()
    pltpu.make_async_copy(rows_bufs[1], out_dst(_STEPS - 1),
                          out_sems[1]).wait()


def kernel(road_ids, road_table, pad_emb, mask_emb):
    table128 = jnp.pad(road_table, ((0, 0), (0, _DP - _D)))
    pm = jnp.pad(jnp.stack([pad_emb, mask_emb]), ((0, 0), (0, _DP - _D)))

    mesh = plsc.VectorSubcoreMesh(core_axis_name="c", subcore_axis_name="s")
    run = pl.kernel(
        _body,
        mesh=mesh,
        compiler_params=pltpu.CompilerParams(needs_layout_passes=False,
                                             use_tc_tiling_on_sc=False),
        out_type=jax.ShapeDtypeStruct((_B, _S, _DP), jnp.float32),
        scratch_types=[
            pltpu.VMEM((_CB, _S), jnp.int32),
            pltpu.VMEM((_CB, _S), jnp.int32),
            pltpu.VMEM((_NSUB, _SUB), jnp.int32),
            pltpu.VMEM((_NSUB, _SUB), jnp.int32),
            pltpu.VMEM((_CB, _S, _DP), jnp.float32),
            pltpu.VMEM((_CB, _S, _DP), jnp.float32),
            pltpu.VMEM((2, _DP), jnp.float32),
            pltpu.SemaphoreType.DMA,
            pltpu.SemaphoreType.DMA,
            pltpu.SemaphoreType.DMA,
            pltpu.SemaphoreType.DMA,
            pltpu.SemaphoreType.DMA,
        ],
    )
    out = run(road_ids, table128, pm)
    return out[:, :, :_D]
